# Initial kernel scaffold; baseline (speedup 1.0000x reference)
import jax
import jax.numpy as jnp
from jax.experimental import pallas as pl


def kernel(input_hr, input_lr, embed_lr, embed_hr):
    raise NotImplementedError("write your pallas kernel here")


# speedup vs baseline: 1.2749x; 1.2749x over previous
import jax
import jax.numpy as jnp
from jax import lax
from jax.experimental import pallas as pl
from jax.experimental.pallas import tpu as pltpu
from jax.experimental.pallas import tpu_sc as plsc

DIM = 32
NE = 8192
M = 16384
BM = 256
NC = 2
NS = 16
NW = NC * NS
BW = M // NW
LANES = 16


def _argmin_body(xl_ref, xh_ref, el_ref, eh_ref, idx_ref):
    x = jnp.concatenate([xl_ref[...], xh_ref[...]], axis=1)
    e = jnp.concatenate([el_ref[...], eh_ref[...]], axis=0)
    s = jnp.sum(x * x, axis=1, keepdims=True)
    b = lax.dot_general(x.astype(jnp.bfloat16), e.astype(jnp.bfloat16),
                        (((1,), (0,)), ((), ())),
                        preferred_element_type=jnp.float32)
    c = jnp.sum(e * e, axis=0, keepdims=True)
    dist = s - 2.0 * b + c
    mn = jnp.min(dist, axis=1, keepdims=True)
    ii = lax.broadcasted_iota(jnp.int32, dist.shape, 1)
    idx_ref[...] = jnp.min(jnp.where(dist == mn, ii, jnp.int32(NE)), axis=1)


TW = 128


CH = 256


def _sc_gather_body(idx_hbm, tab_hbm, xl_hbm, xh_hbm,
                    q_out, part_out,
                    idx_v, q_v, xl_v, xh_v, acc_v, sem):
    wid = lax.axis_index("s") * NC + lax.axis_index("c")
    z = jnp.zeros((LANES,), jnp.float32)
    acc_l, acc_h = z, z
    for ch in range(BW // CH):
        base = wid * BW + ch * CH
        pltpu.sync_copy(idx_hbm.at[pl.ds(base, CH)], idx_v)
        pltpu.async_copy(tab_hbm.at[idx_v], q_v, sem).wait()
        pltpu.sync_copy(q_v, q_out.at[pl.ds(base, CH)])
        pltpu.sync_copy(xl_hbm.at[pl.ds(base, CH)], xl_v)
        pltpu.sync_copy(xh_hbm.at[pl.ds(base, CH)], xh_v)

        def body(i, carry):
            a_l, a_h = carry
            dl0 = q_v[i, pl.ds(0, LANES)] - xl_v[i, pl.ds(0, LANES)]
            dl1 = q_v[i, pl.ds(LANES, LANES)] - xl_v[i, pl.ds(LANES, LANES)]
            dh0 = q_v[i, pl.ds(2 * LANES, LANES)] - xh_v[i, pl.ds(0, LANES)]
            dh1 = q_v[i, pl.ds(3 * LANES, LANES)] - xh_v[i, pl.ds(LANES, LANES)]
            return (a_l + dl0 * dl0 + dl1 * dl1,
                    a_h + dh0 * dh0 + dh1 * dh1)

        acc_l, acc_h = lax.fori_loop(0, CH, body, (acc_l, acc_h))
    acc_v[0, :] = acc_l
    acc_v[1, :] = acc_h
    pltpu.sync_copy(acc_v, part_out.at[wid])


def kernel(input_hr, input_lr, embed_lr, embed_hr):
    x_lr = input_lr.reshape(M, DIM)
    x_hr = input_hr.reshape(M, DIM)

    idx = pl.pallas_call(
        _argmin_body,
        grid=(M // BM,),
        in_specs=[
            pl.BlockSpec((BM, DIM), lambda i: (i, 0)),
            pl.BlockSpec((BM, DIM), lambda i: (i, 0)),
            pl.BlockSpec((DIM, NE), lambda i: (0, 0)),
            pl.BlockSpec((DIM, NE), lambda i: (0, 0)),
        ],
        out_specs=pl.BlockSpec((BM,), lambda i: (i,)),
        out_shape=jax.ShapeDtypeStruct((M,), jnp.int32),
    )(x_lr, x_hr, embed_lr, embed_hr)

    tab = jnp.concatenate(
        [embed_lr.T, embed_hr.T,
         jnp.zeros((NE, TW - 2 * DIM), jnp.float32)], axis=1)

    sc_gather = pl.kernel(
        _sc_gather_body,
        out_type=(
            jax.ShapeDtypeStruct((M, TW), jnp.float32),
            jax.ShapeDtypeStruct((NW, 2, LANES), jnp.float32),
        ),
        mesh=plsc.VectorSubcoreMesh(core_axis_name="c", subcore_axis_name="s"),
        scratch_types=[
            pltpu.VMEM((CH,), jnp.int32),
            pltpu.VMEM((CH, TW), jnp.float32),
            pltpu.VMEM((CH, DIM), jnp.float32),
            pltpu.VMEM((CH, DIM), jnp.float32),
            pltpu.VMEM((2, LANES), jnp.float32),
            pltpu.SemaphoreType.DMA,
        ],
    )
    q_pad, part = sc_gather(idx, tab, x_lr, x_hr)
    q_lr = q_pad[:, :DIM]
    q_hr = q_pad[:, DIM:2 * DIM]

    inv = jnp.float32(1.0 / (M * DIM))
    diff_lr = jnp.sum(part[:, 0, :]) * inv
    diff_hr = jnp.sum(part[:, 1, :]) * inv
    quantize_lr = (x_lr + (q_lr - x_lr)).reshape(input_lr.shape)
    quantize_hr = (x_hr + (q_hr - x_hr)).reshape(input_hr.shape)
    ind = idx.reshape(input_hr.shape[:-1])
    return (quantize_hr, quantize_lr, diff_hr, diff_lr, ind, ind)
